# Initial kernel scaffold; baseline (speedup 1.0000x reference)
import jax
import jax.numpy as jnp
from jax.experimental import pallas as pl


def kernel(embedding_user, embedding_item, edge_weight, user_noise, item_noise, alphas_cumprod, edge_index, user, pos, ts):
    raise NotImplementedError("write your pallas kernel here")


# speedup vs baseline: 1.5939x; 1.5939x over previous
import jax
import jax.numpy as jnp
from jax import lax
from jax.experimental import pallas as pl
from jax.experimental.pallas import tpu as pltpu
from jax.experimental.pallas import tpu_sc as plsc

USER_N = 25000
NODES = 50000
EMB = 64
EDGES = 800000
BATCH = 4096
LAYERS = 3

NC = 2
NS = 16
HALF = 25000
DUMMY = 88
ACC_ROWS = HALF + DUMMY
ACC_TILE = ACC_ROWS // NS
W = 128
E_PAD = 819200
E_TILE = E_PAD // NS
N_WIN = E_TILE // W
PB = BATCH // (NC * NS)
ZR = 112
BR = 1000

_F32 = jnp.float32
_I32 = jnp.int32


def _prop_body(x_hbm, epack_hbm, wpack_hbm, acc_out,
               ep_v, src_v, dst_v, w_v, rows_v, acc_sh, zbuf, sem):
    c = lax.axis_index("c")
    s = lax.axis_index("s")
    base = c * HALF

    zero16 = jnp.zeros((16,), _F32)

    def zrow(i, carry):
        for q in range(4):
            zbuf[i, pl.ds(q * 16, 16)] = zero16
        return carry

    lax.fori_loop(0, ZR, zrow, 0)

    def zcp(k, carry):
        pltpu.sync_copy(zbuf, acc_sh.at[pl.ds(s * ACC_TILE + k * ZR, ZR)])
        return carry

    lax.fori_loop(0, ACC_TILE // ZR, zcp, 0)
    plsc.subcore_barrier()

    iota16 = lax.iota(_I32, 16)

    def win(i, carry):
        wi = s * N_WIN + i
        pltpu.sync_copy(epack_hbm.at[wi], ep_v)
        pltpu.sync_copy(wpack_hbm.at[wi], w_v)
        for k in range(W // 16):
            sl = pl.ds(k * 16, 16)
            src_v[sl] = ep_v[0, sl]
            d = ep_v[1, sl] - base
            ok = (d >= 0) & (d < HALF)
            dummy = HALF + ((k * 16 + iota16) % DUMMY)
            dst_v[sl] = jnp.where(ok, d, dummy)
        pltpu.async_copy(x_hbm.at[src_v], rows_v, sem).wait()

        def rowm(k, carry2):
            wch = w_v[pl.ds(k * 16, 16)]
            for r in range(16):
                e = k * 16 + r
                wv = wch[r]
                for q in range(4):
                    sl = pl.ds(q * 16, 16)
                    rows_v[e, sl] = rows_v[e, sl] * wv
            return carry2

        lax.fori_loop(0, W // 16, rowm, 0)
        pltpu.sync_copy(rows_v, acc_sh.at[dst_v], add=True)
        return carry

    lax.fori_loop(0, N_WIN, win, 0)
    plsc.subcore_barrier()

    @pl.when(s < NS - 1)
    def _():
        pltpu.sync_copy(acc_sh.at[pl.ds(s * ACC_TILE, ACC_TILE)],
                        acc_out.at[pl.ds(base + s * ACC_TILE, ACC_TILE)])

    @pl.when(s == NS - 1)
    def _():
        last = HALF - (NS - 1) * ACC_TILE
        pltpu.sync_copy(acc_sh.at[pl.ds((NS - 1) * ACC_TILE, last)],
                        acc_out.at[pl.ds(base + (NS - 1) * ACC_TILE, last)])


_prop = pl.kernel(
    _prop_body,
    out_type=[
        jax.ShapeDtypeStruct((NODES, EMB), _F32),
    ],
    mesh=plsc.VectorSubcoreMesh(core_axis_name="c", subcore_axis_name="s"),
    compiler_params=pltpu.CompilerParams(use_tc_tiling_on_sc=False),
    scratch_types=[
        pltpu.VMEM((2, W), _I32),
        pltpu.VMEM((W,), _I32),
        pltpu.VMEM((W,), _I32),
        pltpu.VMEM((W,), _F32),
        pltpu.VMEM((W, EMB), _F32),
        pltpu.VMEM_SHARED((ACC_ROWS, EMB), _F32),
        pltpu.VMEM((ZR, EMB), _F32),
        pltpu.SemaphoreType.DMA,
    ],
)


def _norm_body(acc_ref, sum_in_ref, x_out_ref, sum_out_ref):
    x = acc_ref[...]
    s2 = jnp.sum(x * x, axis=1, keepdims=True)
    xn = x / jnp.maximum(jnp.sqrt(s2), _F32(1e-12))
    x_out_ref[...] = xn
    sum_out_ref[...] = sum_in_ref[...] + xn


def _norm(acc, sum_in):
    return pl.pallas_call(
        _norm_body,
        grid=(NODES // BR,),
        in_specs=[
            pl.BlockSpec((BR, EMB), lambda i: (i, 0)),
            pl.BlockSpec((BR, EMB), lambda i: (i, 0)),
        ],
        out_specs=[
            pl.BlockSpec((BR, EMB), lambda i: (i, 0)),
            pl.BlockSpec((BR, EMB), lambda i: (i, 0)),
        ],
        out_shape=[
            jax.ShapeDtypeStruct((NODES, EMB), _F32),
            jax.ShapeDtypeStruct((NODES, EMB), _F32),
        ],
    )(acc, sum_in)


def _final_body(sum_hbm, u_hbm, p_hbm, t_hbm, un_hbm, in_hbm, ta_hbm, tb_hbm,
                nu_out, ni_out,
                idx_v, ts_v, emb_v, noi_v, ca_v, cb_v, tbl_a, tbl_b, sem):
    c = lax.axis_index("c")
    s = lax.axis_index("s")
    wid = s * NC + c
    off = wid * PB

    pltpu.sync_copy(ta_hbm, tbl_a)
    pltpu.sync_copy(tb_hbm, tbl_b)
    pltpu.sync_copy(t_hbm.at[pl.ds(off, PB)], ts_v)
    tav = tbl_a[pl.ds(0, 16)]
    tbv = tbl_b[pl.ds(0, 16)]
    for k in range(PB // 16):
        sl = pl.ds(k * 16, 16)
        tt = ts_v[sl]
        ca = jnp.zeros((16,), _F32)
        cb = jnp.zeros((16,), _F32)
        for step in range(5):
            is_k = tt == step
            ca = jnp.where(is_k, tav[step], ca)
            cb = jnp.where(is_k, tbv[step], cb)
        ca_v[sl] = ca
        cb_v[sl] = cb

    def noise_rows(carry):
        def row(k, carry2):
            ach = ca_v[pl.ds(k * 16, 16)]
            bch = cb_v[pl.ds(k * 16, 16)]
            for r in range(16):
                e = k * 16 + r
                a = ach[r]
                b = bch[r]
                for q in range(4):
                    sl = pl.ds(q * 16, 16)
                    emb_v[e, sl] = emb_v[e, sl] * a + noi_v[e, sl] * b
            return carry2
        return lax.fori_loop(0, PB // 16, row, carry)

    pltpu.sync_copy(u_hbm.at[pl.ds(off, PB)], idx_v)
    pltpu.async_copy(sum_hbm.at[idx_v], emb_v, sem).wait()
    pltpu.sync_copy(un_hbm.at[pl.ds(off, PB)], noi_v)
    noise_rows(0)
    pltpu.sync_copy(emb_v, nu_out.at[pl.ds(off, PB)])

    pltpu.sync_copy(p_hbm.at[pl.ds(off, PB)], idx_v)
    for k in range(PB // 16):
        sl = pl.ds(k * 16, 16)
        idx_v[sl] = idx_v[sl] + USER_N
    pltpu.async_copy(sum_hbm.at[idx_v], emb_v, sem).wait()
    pltpu.sync_copy(in_hbm.at[pl.ds(off, PB)], noi_v)
    noise_rows(0)
    pltpu.sync_copy(emb_v, ni_out.at[pl.ds(off, PB)])


_final = pl.kernel(
    _final_body,
    out_type=[
        jax.ShapeDtypeStruct((BATCH, EMB), _F32),
        jax.ShapeDtypeStruct((BATCH, EMB), _F32),
    ],
    mesh=plsc.VectorSubcoreMesh(core_axis_name="c", subcore_axis_name="s"),
    compiler_params=pltpu.CompilerParams(use_tc_tiling_on_sc=False),
    scratch_types=[
        pltpu.VMEM((PB,), _I32),
        pltpu.VMEM((PB,), _I32),
        pltpu.VMEM((PB, EMB), _F32),
        pltpu.VMEM((PB, EMB), _F32),
        pltpu.VMEM((PB,), _F32),
        pltpu.VMEM((PB,), _F32),
        pltpu.VMEM((16,), _F32),
        pltpu.VMEM((16,), _F32),
        pltpu.SemaphoreType.DMA,
    ],
)


def kernel(embedding_user, embedding_item, edge_weight, user_noise,
           item_noise, alphas_cumprod, edge_index, user, pos, ts):
    all_emb = jnp.concatenate([embedding_user, embedding_item], axis=0)
    pad = E_PAD - EDGES
    src_p = jnp.concatenate([edge_index[0], jnp.zeros((pad,), _I32)])
    dst_p = jnp.concatenate([edge_index[1], jnp.full((pad,), NODES, _I32)])
    w_p = jnp.concatenate([edge_weight, jnp.zeros((pad,), _F32)])
    epack = jnp.stack([src_p.reshape(-1, W), dst_p.reshape(-1, W)], axis=1)
    wpack = w_p.reshape(-1, W)

    ta = jnp.pad(jnp.sqrt(alphas_cumprod), (0, 11))
    tb = jnp.pad(jnp.sqrt(1.0 - alphas_cumprod), (0, 11))

    x = all_emb
    ssum = all_emb
    for _ in range(LAYERS):
        (acc,) = _prop(x, epack, wpack)
        x, ssum = _norm(acc, ssum)

    nu, ni = _final(ssum, user, pos, ts, user_noise, item_noise, ta, tb)
    items = ssum[USER_N:]
    return nu, ni, items


# speedup vs baseline: 2.3263x; 1.4595x over previous
import jax
import jax.numpy as jnp
from jax import lax
from jax.experimental import pallas as pl
from jax.experimental.pallas import tpu as pltpu
from jax.experimental.pallas import tpu_sc as plsc

USER_N = 25000
NODES = 50000
EMB = 64
EDGES = 800000
BATCH = 4096
LAYERS = 3

NC = 2
NS = 16
HALF = 25000
DUMMY = 88
ACC_ROWS = HALF + DUMMY
ACC_TILE = ACC_ROWS // NS
W = 128
E_PAD = 823296
E_TILE = E_PAD // NS
N_WIN = E_TILE // W
PB = BATCH // (NC * NS)
ZR = 112
BR = 1000

_F32 = jnp.float32
_I32 = jnp.int32


NB = 3
N_SUPER = N_WIN // NB


def _prop_body(x_hbm, epack_hbm, wpack_hbm, acc_out,
               ep_v, dst_v, w_v, rows_v, acc_sh,
               esem0, esem1, esem2,
               wsem0, wsem1, wsem2,
               gsem0, gsem1, gsem2,
               ssem0, ssem1, ssem2):
    esem = [esem0, esem1, esem2]
    wsem = [wsem0, wsem1, wsem2]
    gsem = [gsem0, gsem1, gsem2]
    ssem = [ssem0, ssem1, ssem2]

    c = lax.axis_index("c")
    s = lax.axis_index("s")
    base = c * HALF

    zero16 = jnp.zeros((16,), _F32)

    def zrow(i, carry):
        for q in range(4):
            rows_v[0, i, pl.ds(q * 16, 16)] = zero16
        return carry

    lax.fori_loop(0, ZR, zrow, 0)

    def zcp(k, carry):
        pltpu.sync_copy(rows_v.at[0, pl.ds(0, ZR)],
                        acc_sh.at[pl.ds(s * ACC_TILE + k * ZR, ZR)])
        return carry

    lax.fori_loop(0, ACC_TILE // ZR, zcp, 0)
    plsc.subcore_barrier()

    iota16 = lax.iota(_I32, 16)

    def start_lin(wi, b):
        pltpu.async_copy(epack_hbm.at[s * N_WIN + wi], ep_v.at[b], esem[b])
        pltpu.async_copy(wpack_hbm.at[s * N_WIN + wi], w_v.at[b], wsem[b])

    def wait_lin(b):
        pltpu.make_async_copy(epack_hbm.at[0], ep_v.at[b], esem[b]).wait()
        pltpu.make_async_copy(wpack_hbm.at[0], w_v.at[b], wsem[b]).wait()

    def start_gather(b):
        pltpu.async_copy(x_hbm.at[ep_v.at[b, 0]], rows_v.at[b], gsem[b])

    def wait_gather(b):
        pltpu.make_async_copy(x_hbm.at[ep_v.at[b, 0]], rows_v.at[b],
                              gsem[b]).wait()

    def start_scatter(b):
        pltpu.async_copy(rows_v.at[b], acc_sh.at[dst_v.at[b]], ssem[b],
                         add=True)

    def wait_scatter(b):
        pltpu.make_async_copy(rows_v.at[b], acc_sh.at[dst_v.at[b]],
                              ssem[b]).wait()

    for b in range(NB):
        start_lin(b, b)

    def super_step(g, carry):
        for b in range(NB):
            wait_lin(b)

            @pl.when(g > 0)
            def _():
                wait_scatter(b)

            for k in range(W // 16):
                sl = pl.ds(k * 16, 16)
                d = ep_v[b, 1, sl] - base
                ok = (d >= 0) & (d < HALF)
                dummy = HALF + ((k * 16 + iota16) % DUMMY)
                dst_v[b, sl] = jnp.where(ok, d, dummy)
            start_gather(b)
        for b in range(NB):
            wait_gather(b)

            def rowm(k, carry2):
                wch = w_v[b, pl.ds(k * 16, 16)]
                for r in range(16):
                    e = k * 16 + r
                    wv = wch[r]
                    for q in range(4):
                        sl = pl.ds(q * 16, 16)
                        rows_v[b, e, sl] = rows_v[b, e, sl] * wv
                return carry2

            lax.fori_loop(0, W // 16, rowm, 0)
            start_scatter(b)

            @pl.when(g < N_SUPER - 1)
            def _():
                start_lin(g * NB + b + NB, b)
        return carry

    lax.fori_loop(0, N_SUPER, super_step, 0)
    for b in range(NB):
        wait_scatter(b)
    plsc.subcore_barrier()

    @pl.when(s < NS - 1)
    def _():
        pltpu.sync_copy(acc_sh.at[pl.ds(s * ACC_TILE, ACC_TILE)],
                        acc_out.at[pl.ds(base + s * ACC_TILE, ACC_TILE)])

    @pl.when(s == NS - 1)
    def _():
        last = HALF - (NS - 1) * ACC_TILE
        pltpu.sync_copy(acc_sh.at[pl.ds((NS - 1) * ACC_TILE, last)],
                        acc_out.at[pl.ds(base + (NS - 1) * ACC_TILE, last)])


_prop = pl.kernel(
    _prop_body,
    out_type=[
        jax.ShapeDtypeStruct((NODES, EMB), _F32),
    ],
    mesh=plsc.VectorSubcoreMesh(core_axis_name="c", subcore_axis_name="s"),
    compiler_params=pltpu.CompilerParams(use_tc_tiling_on_sc=False),
    scratch_types=[
        pltpu.VMEM((NB, 2, W), _I32),
        pltpu.VMEM((NB, W), _I32),
        pltpu.VMEM((NB, W), _F32),
        pltpu.VMEM((NB, W, EMB), _F32),
        pltpu.VMEM_SHARED((ACC_ROWS, EMB), _F32),
    ] + [pltpu.SemaphoreType.DMA] * (4 * NB),
)


def _norm_body(acc_ref, sum_in_ref, x_out_ref, sum_out_ref):
    x = acc_ref[...]
    s2 = jnp.sum(x * x, axis=1, keepdims=True)
    xn = x / jnp.maximum(jnp.sqrt(s2), _F32(1e-12))
    x_out_ref[...] = xn
    sum_out_ref[...] = sum_in_ref[...] + xn


def _norm(acc, sum_in):
    return pl.pallas_call(
        _norm_body,
        grid=(NODES // BR,),
        in_specs=[
            pl.BlockSpec((BR, EMB), lambda i: (i, 0)),
            pl.BlockSpec((BR, EMB), lambda i: (i, 0)),
        ],
        out_specs=[
            pl.BlockSpec((BR, EMB), lambda i: (i, 0)),
            pl.BlockSpec((BR, EMB), lambda i: (i, 0)),
        ],
        out_shape=[
            jax.ShapeDtypeStruct((NODES, EMB), _F32),
            jax.ShapeDtypeStruct((NODES, EMB), _F32),
        ],
    )(acc, sum_in)


def _final_body(sum_hbm, u_hbm, p_hbm, t_hbm, un_hbm, in_hbm, ta_hbm, tb_hbm,
                nu_out, ni_out,
                idx_v, ts_v, emb_v, noi_v, ca_v, cb_v, tbl_a, tbl_b, sem):
    c = lax.axis_index("c")
    s = lax.axis_index("s")
    wid = s * NC + c
    off = wid * PB

    pltpu.sync_copy(ta_hbm, tbl_a)
    pltpu.sync_copy(tb_hbm, tbl_b)
    pltpu.sync_copy(t_hbm.at[pl.ds(off, PB)], ts_v)
    tav = tbl_a[pl.ds(0, 16)]
    tbv = tbl_b[pl.ds(0, 16)]
    for k in range(PB // 16):
        sl = pl.ds(k * 16, 16)
        tt = ts_v[sl]
        ca = jnp.zeros((16,), _F32)
        cb = jnp.zeros((16,), _F32)
        for step in range(5):
            is_k = tt == step
            ca = jnp.where(is_k, tav[step], ca)
            cb = jnp.where(is_k, tbv[step], cb)
        ca_v[sl] = ca
        cb_v[sl] = cb

    def noise_rows(carry):
        def row(k, carry2):
            ach = ca_v[pl.ds(k * 16, 16)]
            bch = cb_v[pl.ds(k * 16, 16)]
            for r in range(16):
                e = k * 16 + r
                a = ach[r]
                b = bch[r]
                for q in range(4):
                    sl = pl.ds(q * 16, 16)
                    emb_v[e, sl] = emb_v[e, sl] * a + noi_v[e, sl] * b
            return carry2
        return lax.fori_loop(0, PB // 16, row, carry)

    pltpu.sync_copy(u_hbm.at[pl.ds(off, PB)], idx_v)
    pltpu.async_copy(sum_hbm.at[idx_v], emb_v, sem).wait()
    pltpu.sync_copy(un_hbm.at[pl.ds(off, PB)], noi_v)
    noise_rows(0)
    pltpu.sync_copy(emb_v, nu_out.at[pl.ds(off, PB)])

    pltpu.sync_copy(p_hbm.at[pl.ds(off, PB)], idx_v)
    for k in range(PB // 16):
        sl = pl.ds(k * 16, 16)
        idx_v[sl] = idx_v[sl] + USER_N
    pltpu.async_copy(sum_hbm.at[idx_v], emb_v, sem).wait()
    pltpu.sync_copy(in_hbm.at[pl.ds(off, PB)], noi_v)
    noise_rows(0)
    pltpu.sync_copy(emb_v, ni_out.at[pl.ds(off, PB)])


_final = pl.kernel(
    _final_body,
    out_type=[
        jax.ShapeDtypeStruct((BATCH, EMB), _F32),
        jax.ShapeDtypeStruct((BATCH, EMB), _F32),
    ],
    mesh=plsc.VectorSubcoreMesh(core_axis_name="c", subcore_axis_name="s"),
    compiler_params=pltpu.CompilerParams(use_tc_tiling_on_sc=False),
    scratch_types=[
        pltpu.VMEM((PB,), _I32),
        pltpu.VMEM((PB,), _I32),
        pltpu.VMEM((PB, EMB), _F32),
        pltpu.VMEM((PB, EMB), _F32),
        pltpu.VMEM((PB,), _F32),
        pltpu.VMEM((PB,), _F32),
        pltpu.VMEM((16,), _F32),
        pltpu.VMEM((16,), _F32),
        pltpu.SemaphoreType.DMA,
    ],
)


def kernel(embedding_user, embedding_item, edge_weight, user_noise,
           item_noise, alphas_cumprod, edge_index, user, pos, ts):
    all_emb = jnp.concatenate([embedding_user, embedding_item], axis=0)
    pad = E_PAD - EDGES
    src_p = jnp.concatenate([edge_index[0], jnp.zeros((pad,), _I32)])
    dst_p = jnp.concatenate([edge_index[1], jnp.full((pad,), NODES, _I32)])
    w_p = jnp.concatenate([edge_weight, jnp.zeros((pad,), _F32)])
    epack = jnp.stack([src_p.reshape(-1, W), dst_p.reshape(-1, W)], axis=1)
    wpack = w_p.reshape(-1, W)

    ta = jnp.pad(jnp.sqrt(alphas_cumprod), (0, 11))
    tb = jnp.pad(jnp.sqrt(1.0 - alphas_cumprod), (0, 11))

    x = all_emb
    ssum = all_emb
    for _ in range(LAYERS):
        (acc,) = _prop(x, epack, wpack)
        x, ssum = _norm(acc, ssum)

    nu, ni = _final(ssum, user, pos, ts, user_noise, item_noise, ta, tb)
    items = ssum[USER_N:]
    return nu, ni, items


# speedup vs baseline: 4.8888x; 2.1015x over previous
import jax
import jax.numpy as jnp
from jax import lax
from jax.experimental import pallas as pl
from jax.experimental.pallas import tpu as pltpu
from jax.experimental.pallas import tpu_sc as plsc

USER_N = 25000
NODES = 50000
EMB = 64
HEMB = EMB // 2
EDGES = 800000
BATCH = 4096
LAYERS = 3

NC = 2
NS = 16
PADR = 48
ACC_ROWS = NODES + PADR
ACC_TILE = ACC_ROWS // NS
W = 128
NB = 3
N_WIN = 402
E_TILE = N_WIN * W
E_PAD = E_TILE * NS
N_SUPER = N_WIN // NB
PB = BATCH // (NC * NS)
BR = 1000

_F32 = jnp.float32
_I32 = jnp.int32


def _prop_body(x2_hbm, epack_hbm, wpack_hbm, acc_out,
               ep_v, dst_v, w_v, rows_v, acc_sh,
               esem0, esem1, esem2,
               wsem0, wsem1, wsem2,
               gsem0, gsem1, gsem2,
               ssem0, ssem1, ssem2):
    esem = [esem0, esem1, esem2]
    wsem = [wsem0, wsem1, wsem2]
    gsem = [gsem0, gsem1, gsem2]
    ssem = [ssem0, ssem1, ssem2]

    c = lax.axis_index("c")
    s = lax.axis_index("s")

    zero16 = jnp.zeros((16,), _F32)

    def zrow(i, carry):
        for q in range(HEMB // 16):
            rows_v[0, i, pl.ds(q * 16, 16)] = zero16
        return carry

    lax.fori_loop(0, W, zrow, 0)

    def zcp(k, carry):
        pltpu.sync_copy(rows_v.at[0],
                        acc_sh.at[pl.ds(s * ACC_TILE + k * W, W)])
        return carry

    lax.fori_loop(0, ACC_TILE // W, zcp, 0)
    pltpu.sync_copy(rows_v.at[0, pl.ds(0, ACC_TILE - (ACC_TILE // W) * W)],
                    acc_sh.at[pl.ds(s * ACC_TILE + (ACC_TILE // W) * W,
                                    ACC_TILE - (ACC_TILE // W) * W)])
    plsc.subcore_barrier()

    def start_lin(wi, b):
        pltpu.async_copy(epack_hbm.at[s * N_WIN + wi], ep_v.at[b], esem[b])
        pltpu.async_copy(wpack_hbm.at[s * N_WIN + wi], w_v.at[b], wsem[b])

    def wait_lin(b):
        pltpu.make_async_copy(epack_hbm.at[0], ep_v.at[b], esem[b]).wait()
        pltpu.make_async_copy(wpack_hbm.at[0], w_v.at[b], wsem[b]).wait()

    def start_gather(b):
        pltpu.async_copy(x2_hbm.at[c].at[ep_v.at[b, 0]], rows_v.at[b],
                         gsem[b])

    def wait_gather(b):
        pltpu.make_async_copy(x2_hbm.at[c].at[ep_v.at[b, 0]], rows_v.at[b],
                              gsem[b]).wait()

    def start_scatter(b):
        pltpu.async_copy(rows_v.at[b], acc_sh.at[dst_v.at[b]], ssem[b],
                         add=True)

    def wait_scatter(b):
        pltpu.make_async_copy(rows_v.at[b], acc_sh.at[dst_v.at[b]],
                              ssem[b]).wait()

    for b in range(NB):
        start_lin(b, b)

    def super_step(g, carry):
        for b in range(NB):
            wait_lin(b)

            @pl.when(g > 0)
            def _():
                wait_scatter(b)

            for k in range(W // 16):
                sl = pl.ds(k * 16, 16)
                dst_v[b, sl] = ep_v[b, 1, sl]
            start_gather(b)
        for b in range(NB):
            wait_gather(b)

            def rowm(k, carry2):
                wch = w_v[b, pl.ds(k * 16, 16)]
                for r in range(16):
                    e = k * 16 + r
                    wv = wch[r]
                    for q in range(HEMB // 16):
                        sl = pl.ds(q * 16, 16)
                        rows_v[b, e, sl] = rows_v[b, e, sl] * wv
                return carry2

            lax.fori_loop(0, W // 16, rowm, 0)
            start_scatter(b)

            @pl.when(g < N_SUPER - 1)
            def _():
                start_lin(g * NB + b + NB, b)
        return carry

    lax.fori_loop(0, N_SUPER, super_step, 0)
    for b in range(NB):
        wait_scatter(b)
    plsc.subcore_barrier()

    @pl.when(s < NS - 1)
    def _():
        pltpu.sync_copy(acc_sh.at[pl.ds(s * ACC_TILE, ACC_TILE)],
                        acc_out.at[c].at[pl.ds(s * ACC_TILE, ACC_TILE)])

    @pl.when(s == NS - 1)
    def _():
        last = NODES - (NS - 1) * ACC_TILE
        pltpu.sync_copy(acc_sh.at[pl.ds((NS - 1) * ACC_TILE, last)],
                        acc_out.at[c].at[pl.ds((NS - 1) * ACC_TILE, last)])


_prop = pl.kernel(
    _prop_body,
    out_type=[
        jax.ShapeDtypeStruct((NC, NODES, HEMB), _F32),
    ],
    mesh=plsc.VectorSubcoreMesh(core_axis_name="c", subcore_axis_name="s"),
    compiler_params=pltpu.CompilerParams(use_tc_tiling_on_sc=False),
    scratch_types=[
        pltpu.VMEM((NB, 2, W), _I32),
        pltpu.VMEM((NB, W), _I32),
        pltpu.VMEM((NB, W), _F32),
        pltpu.VMEM((NB, W, HEMB), _F32),
        pltpu.VMEM_SHARED((ACC_ROWS, HEMB), _F32),
    ] + [pltpu.SemaphoreType.DMA] * (4 * NB),
)


def _norm_body(acc_ref, sum_in_ref, x2_out_ref, sum_out_ref):
    lo = acc_ref[0]
    hi = acc_ref[1]
    x = jnp.concatenate([lo, hi], axis=1)
    s2 = jnp.sum(x * x, axis=1, keepdims=True)
    inv = _F32(1.0) / jnp.maximum(jnp.sqrt(s2), _F32(1e-12))
    x2_out_ref[0] = lo * inv
    x2_out_ref[1] = hi * inv
    sum_out_ref[...] = sum_in_ref[...] + x * inv


def _norm(acc2, sum_in):
    return pl.pallas_call(
        _norm_body,
        grid=(NODES // BR,),
        in_specs=[
            pl.BlockSpec((NC, BR, HEMB), lambda i: (0, i, 0)),
            pl.BlockSpec((BR, EMB), lambda i: (i, 0)),
        ],
        out_specs=[
            pl.BlockSpec((NC, BR, HEMB), lambda i: (0, i, 0)),
            pl.BlockSpec((BR, EMB), lambda i: (i, 0)),
        ],
        out_shape=[
            jax.ShapeDtypeStruct((NC, NODES, HEMB), _F32),
            jax.ShapeDtypeStruct((NODES, EMB), _F32),
        ],
    )(acc2, sum_in)


def _final_body(sum_hbm, u_hbm, p_hbm, t_hbm, un_hbm, in_hbm, ta_hbm, tb_hbm,
                nu_out, ni_out,
                idx_v, ts_v, emb_v, noi_v, ca_v, cb_v, tbl_a, tbl_b, sem):
    c = lax.axis_index("c")
    s = lax.axis_index("s")
    wid = s * NC + c
    off = wid * PB

    pltpu.sync_copy(ta_hbm, tbl_a)
    pltpu.sync_copy(tb_hbm, tbl_b)
    pltpu.sync_copy(t_hbm.at[pl.ds(off, PB)], ts_v)
    tav = tbl_a[pl.ds(0, 16)]
    tbv = tbl_b[pl.ds(0, 16)]
    for k in range(PB // 16):
        sl = pl.ds(k * 16, 16)
        tt = ts_v[sl]
        ca = jnp.zeros((16,), _F32)
        cb = jnp.zeros((16,), _F32)
        for step in range(5):
            is_k = tt == step
            ca = jnp.where(is_k, tav[step], ca)
            cb = jnp.where(is_k, tbv[step], cb)
        ca_v[sl] = ca
        cb_v[sl] = cb

    def noise_rows(carry):
        def row(k, carry2):
            ach = ca_v[pl.ds(k * 16, 16)]
            bch = cb_v[pl.ds(k * 16, 16)]
            for r in range(16):
                e = k * 16 + r
                a = ach[r]
                b = bch[r]
                for q in range(4):
                    sl = pl.ds(q * 16, 16)
                    emb_v[e, sl] = emb_v[e, sl] * a + noi_v[e, sl] * b
            return carry2
        return lax.fori_loop(0, PB // 16, row, carry)

    pltpu.sync_copy(u_hbm.at[pl.ds(off, PB)], idx_v)
    pltpu.async_copy(sum_hbm.at[idx_v], emb_v, sem).wait()
    pltpu.sync_copy(un_hbm.at[pl.ds(off, PB)], noi_v)
    noise_rows(0)
    pltpu.sync_copy(emb_v, nu_out.at[pl.ds(off, PB)])

    pltpu.sync_copy(p_hbm.at[pl.ds(off, PB)], idx_v)
    for k in range(PB // 16):
        sl = pl.ds(k * 16, 16)
        idx_v[sl] = idx_v[sl] + USER_N
    pltpu.async_copy(sum_hbm.at[idx_v], emb_v, sem).wait()
    pltpu.sync_copy(in_hbm.at[pl.ds(off, PB)], noi_v)
    noise_rows(0)
    pltpu.sync_copy(emb_v, ni_out.at[pl.ds(off, PB)])


_final = pl.kernel(
    _final_body,
    out_type=[
        jax.ShapeDtypeStruct((BATCH, EMB), _F32),
        jax.ShapeDtypeStruct((BATCH, EMB), _F32),
    ],
    mesh=plsc.VectorSubcoreMesh(core_axis_name="c", subcore_axis_name="s"),
    compiler_params=pltpu.CompilerParams(use_tc_tiling_on_sc=False),
    scratch_types=[
        pltpu.VMEM((PB,), _I32),
        pltpu.VMEM((PB,), _I32),
        pltpu.VMEM((PB, EMB), _F32),
        pltpu.VMEM((PB, EMB), _F32),
        pltpu.VMEM((PB,), _F32),
        pltpu.VMEM((PB,), _F32),
        pltpu.VMEM((16,), _F32),
        pltpu.VMEM((16,), _F32),
        pltpu.SemaphoreType.DMA,
    ],
)


def kernel(embedding_user, embedding_item, edge_weight, user_noise,
           item_noise, alphas_cumprod, edge_index, user, pos, ts):
    all_emb = jnp.concatenate([embedding_user, embedding_item], axis=0)
    pad = E_PAD - EDGES
    src_p = jnp.concatenate([edge_index[0], jnp.zeros((pad,), _I32)])
    dst_p = jnp.concatenate(
        [edge_index[1],
         NODES + (jnp.arange(pad, dtype=_I32) % PADR)])
    w_p = jnp.concatenate([edge_weight, jnp.zeros((pad,), _F32)])
    epack = jnp.stack([src_p.reshape(-1, W), dst_p.reshape(-1, W)], axis=1)
    wpack = w_p.reshape(-1, W)

    ta = jnp.pad(jnp.sqrt(alphas_cumprod), (0, 11))
    tb = jnp.pad(jnp.sqrt(1.0 - alphas_cumprod), (0, 11))

    x2 = jnp.stack([all_emb[:, :HEMB], all_emb[:, HEMB:]], axis=0)
    ssum = all_emb
    for _ in range(LAYERS):
        (acc2,) = _prop(x2, epack, wpack)
        x2, ssum = _norm(acc2, ssum)

    nu, ni = _final(ssum, user, pos, ts, user_noise, item_noise, ta, tb)
    items = ssum[USER_N:]
    return nu, ni, items


# speedup vs baseline: 5.1552x; 1.0545x over previous
import jax
import jax.numpy as jnp
from jax import lax
from jax.experimental import pallas as pl
from jax.experimental.pallas import tpu as pltpu
from jax.experimental.pallas import tpu_sc as plsc

USER_N = 25000
NODES = 50000
EMB = 64
HEMB = EMB // 2
EDGES = 800000
BATCH = 4096
LAYERS = 3

NC = 2
NS = 16
PADR = 48
ACC_ROWS = NODES + PADR
ACC_TILE = ACC_ROWS // NS
W = 128
NB = 6
N_WIN = 402
E_TILE = N_WIN * W
E_PAD = E_TILE * NS
N_SUPER = N_WIN // NB
PB = BATCH // (NC * NS)
BR = 1000

_F32 = jnp.float32
_I32 = jnp.int32


def _prop_body(x2_hbm, epack_hbm, wpack_hbm, acc_out,
               ep_v, dst_v, w_v, rows_v, acc_sh,
               esem0, esem1, esem2, esem3, esem4, esem5,
               wsem0, wsem1, wsem2, wsem3, wsem4, wsem5,
               gsem0, gsem1, gsem2, gsem3, gsem4, gsem5,
               ssem0, ssem1, ssem2, ssem3, ssem4, ssem5):
    esem = [esem0, esem1, esem2, esem3, esem4, esem5]
    wsem = [wsem0, wsem1, wsem2, wsem3, wsem4, wsem5]
    gsem = [gsem0, gsem1, gsem2, gsem3, gsem4, gsem5]
    ssem = [ssem0, ssem1, ssem2, ssem3, ssem4, ssem5]

    c = lax.axis_index("c")
    s = lax.axis_index("s")

    zero16 = jnp.zeros((16,), _F32)

    def zrow(i, carry):
        for q in range(HEMB // 16):
            rows_v[0, i, pl.ds(q * 16, 16)] = zero16
        return carry

    lax.fori_loop(0, W, zrow, 0)

    def zcp(k, carry):
        pltpu.sync_copy(rows_v.at[0],
                        acc_sh.at[pl.ds(s * ACC_TILE + k * W, W)])
        return carry

    lax.fori_loop(0, ACC_TILE // W, zcp, 0)
    pltpu.sync_copy(rows_v.at[0, pl.ds(0, ACC_TILE - (ACC_TILE // W) * W)],
                    acc_sh.at[pl.ds(s * ACC_TILE + (ACC_TILE // W) * W,
                                    ACC_TILE - (ACC_TILE // W) * W)])
    plsc.subcore_barrier()

    def start_lin(wi, b):
        pltpu.async_copy(epack_hbm.at[s * N_WIN + wi], ep_v.at[b], esem[b])
        pltpu.async_copy(wpack_hbm.at[s * N_WIN + wi], w_v.at[b], wsem[b])

    def wait_lin(b):
        pltpu.make_async_copy(epack_hbm.at[0], ep_v.at[b], esem[b]).wait()
        pltpu.make_async_copy(wpack_hbm.at[0], w_v.at[b], wsem[b]).wait()

    def start_gather(b):
        pltpu.async_copy(x2_hbm.at[c].at[ep_v.at[b, 0]], rows_v.at[b],
                         gsem[b])

    def wait_gather(b):
        pltpu.make_async_copy(x2_hbm.at[c].at[ep_v.at[b, 0]], rows_v.at[b],
                              gsem[b]).wait()

    def start_scatter(b):
        pltpu.async_copy(rows_v.at[b], acc_sh.at[dst_v.at[b]], ssem[b],
                         add=True)

    def wait_scatter(b):
        pltpu.make_async_copy(rows_v.at[b], acc_sh.at[dst_v.at[b]],
                              ssem[b]).wait()

    for b in range(NB):
        start_lin(b, b)

    def super_step(g, carry):
        for b in range(NB):
            wait_lin(b)

            @pl.when(g > 0)
            def _():
                wait_scatter(b)

            for k in range(W // 16):
                sl = pl.ds(k * 16, 16)
                dst_v[b, sl] = ep_v[b, 1, sl]
            start_gather(b)
        for b in range(NB):
            wait_gather(b)

            def rowm(k, carry2):
                wch = w_v[b, pl.ds(k * 16, 16)]
                for r in range(16):
                    e = k * 16 + r
                    wv = wch[r]
                    for q in range(HEMB // 16):
                        sl = pl.ds(q * 16, 16)
                        rows_v[b, e, sl] = rows_v[b, e, sl] * wv
                return carry2

            lax.fori_loop(0, W // 16, rowm, 0)
            start_scatter(b)

            @pl.when(g < N_SUPER - 1)
            def _():
                start_lin(g * NB + b + NB, b)
        return carry

    lax.fori_loop(0, N_SUPER, super_step, 0)
    for b in range(NB):
        wait_scatter(b)
    plsc.subcore_barrier()

    @pl.when(s < NS - 1)
    def _():
        pltpu.sync_copy(acc_sh.at[pl.ds(s * ACC_TILE, ACC_TILE)],
                        acc_out.at[c].at[pl.ds(s * ACC_TILE, ACC_TILE)])

    @pl.when(s == NS - 1)
    def _():
        last = NODES - (NS - 1) * ACC_TILE
        pltpu.sync_copy(acc_sh.at[pl.ds((NS - 1) * ACC_TILE, last)],
                        acc_out.at[c].at[pl.ds((NS - 1) * ACC_TILE, last)])


_prop = pl.kernel(
    _prop_body,
    out_type=[
        jax.ShapeDtypeStruct((NC, NODES, HEMB), _F32),
    ],
    mesh=plsc.VectorSubcoreMesh(core_axis_name="c", subcore_axis_name="s"),
    compiler_params=pltpu.CompilerParams(use_tc_tiling_on_sc=False),
    scratch_types=[
        pltpu.VMEM((NB, 2, W), _I32),
        pltpu.VMEM((NB, W), _I32),
        pltpu.VMEM((NB, W), _F32),
        pltpu.VMEM((NB, W, HEMB), _F32),
        pltpu.VMEM_SHARED((ACC_ROWS, HEMB), _F32),
    ] + [pltpu.SemaphoreType.DMA] * (4 * NB),
)


def _norm_body(acc_ref, sum_in_ref, x2_out_ref, sum_out_ref):
    lo = acc_ref[0]
    hi = acc_ref[1]
    x = jnp.concatenate([lo, hi], axis=1)
    s2 = jnp.sum(x * x, axis=1, keepdims=True)
    inv = _F32(1.0) / jnp.maximum(jnp.sqrt(s2), _F32(1e-12))
    x2_out_ref[0] = lo * inv
    x2_out_ref[1] = hi * inv
    sum_out_ref[...] = sum_in_ref[...] + x * inv


def _norm(acc2, sum_in):
    return pl.pallas_call(
        _norm_body,
        grid=(NODES // BR,),
        in_specs=[
            pl.BlockSpec((NC, BR, HEMB), lambda i: (0, i, 0)),
            pl.BlockSpec((BR, EMB), lambda i: (i, 0)),
        ],
        out_specs=[
            pl.BlockSpec((NC, BR, HEMB), lambda i: (0, i, 0)),
            pl.BlockSpec((BR, EMB), lambda i: (i, 0)),
        ],
        out_shape=[
            jax.ShapeDtypeStruct((NC, NODES, HEMB), _F32),
            jax.ShapeDtypeStruct((NODES, EMB), _F32),
        ],
    )(acc2, sum_in)


def _final_body(sum_hbm, u_hbm, p_hbm, t_hbm, un_hbm, in_hbm, ta_hbm, tb_hbm,
                nu_out, ni_out,
                idx_v, ts_v, emb_v, noi_v, ca_v, cb_v, tbl_a, tbl_b, sem):
    c = lax.axis_index("c")
    s = lax.axis_index("s")
    wid = s * NC + c
    off = wid * PB

    pltpu.sync_copy(ta_hbm, tbl_a)
    pltpu.sync_copy(tb_hbm, tbl_b)
    pltpu.sync_copy(t_hbm.at[pl.ds(off, PB)], ts_v)
    tav = tbl_a[pl.ds(0, 16)]
    tbv = tbl_b[pl.ds(0, 16)]
    for k in range(PB // 16):
        sl = pl.ds(k * 16, 16)
        tt = ts_v[sl]
        ca = jnp.zeros((16,), _F32)
        cb = jnp.zeros((16,), _F32)
        for step in range(5):
            is_k = tt == step
            ca = jnp.where(is_k, tav[step], ca)
            cb = jnp.where(is_k, tbv[step], cb)
        ca_v[sl] = ca
        cb_v[sl] = cb

    def noise_rows(carry):
        def row(k, carry2):
            ach = ca_v[pl.ds(k * 16, 16)]
            bch = cb_v[pl.ds(k * 16, 16)]
            for r in range(16):
                e = k * 16 + r
                a = ach[r]
                b = bch[r]
                for q in range(4):
                    sl = pl.ds(q * 16, 16)
                    emb_v[e, sl] = emb_v[e, sl] * a + noi_v[e, sl] * b
            return carry2
        return lax.fori_loop(0, PB // 16, row, carry)

    pltpu.sync_copy(u_hbm.at[pl.ds(off, PB)], idx_v)
    pltpu.async_copy(sum_hbm.at[idx_v], emb_v, sem).wait()
    pltpu.sync_copy(un_hbm.at[pl.ds(off, PB)], noi_v)
    noise_rows(0)
    pltpu.sync_copy(emb_v, nu_out.at[pl.ds(off, PB)])

    pltpu.sync_copy(p_hbm.at[pl.ds(off, PB)], idx_v)
    for k in range(PB // 16):
        sl = pl.ds(k * 16, 16)
        idx_v[sl] = idx_v[sl] + USER_N
    pltpu.async_copy(sum_hbm.at[idx_v], emb_v, sem).wait()
    pltpu.sync_copy(in_hbm.at[pl.ds(off, PB)], noi_v)
    noise_rows(0)
    pltpu.sync_copy(emb_v, ni_out.at[pl.ds(off, PB)])


_final = pl.kernel(
    _final_body,
    out_type=[
        jax.ShapeDtypeStruct((BATCH, EMB), _F32),
        jax.ShapeDtypeStruct((BATCH, EMB), _F32),
    ],
    mesh=plsc.VectorSubcoreMesh(core_axis_name="c", subcore_axis_name="s"),
    compiler_params=pltpu.CompilerParams(use_tc_tiling_on_sc=False),
    scratch_types=[
        pltpu.VMEM((PB,), _I32),
        pltpu.VMEM((PB,), _I32),
        pltpu.VMEM((PB, EMB), _F32),
        pltpu.VMEM((PB, EMB), _F32),
        pltpu.VMEM((PB,), _F32),
        pltpu.VMEM((PB,), _F32),
        pltpu.VMEM((16,), _F32),
        pltpu.VMEM((16,), _F32),
        pltpu.SemaphoreType.DMA,
    ],
)


def kernel(embedding_user, embedding_item, edge_weight, user_noise,
           item_noise, alphas_cumprod, edge_index, user, pos, ts):
    all_emb = jnp.concatenate([embedding_user, embedding_item], axis=0)
    pad = E_PAD - EDGES
    src_p = jnp.concatenate([edge_index[0], jnp.zeros((pad,), _I32)])
    dst_p = jnp.concatenate(
        [edge_index[1],
         NODES + (jnp.arange(pad, dtype=_I32) % PADR)])
    w_p = jnp.concatenate([edge_weight, jnp.zeros((pad,), _F32)])
    epack = jnp.stack([src_p.reshape(-1, W), dst_p.reshape(-1, W)], axis=1)
    wpack = w_p.reshape(-1, W)

    ta = jnp.pad(jnp.sqrt(alphas_cumprod), (0, 11))
    tb = jnp.pad(jnp.sqrt(1.0 - alphas_cumprod), (0, 11))

    x2 = jnp.stack([all_emb[:, :HEMB], all_emb[:, HEMB:]], axis=0)
    ssum = all_emb
    for _ in range(LAYERS):
        (acc2,) = _prop(x2, epack, wpack)
        x2, ssum = _norm(acc2, ssum)

    nu, ni = _final(ssum, user, pos, ts, user_noise, item_noise, ta, tb)
    items = ssum[USER_N:]
    return nu, ni, items
